# baseline (speedup 1.0000x reference)
import functools

import jax
import jax.numpy as jnp
from jax import lax
from jax.experimental import pallas as pl
from jax.experimental.pallas import tpu as pltpu
from jax.experimental.pallas import tpu_sc as plsc

NUM_CORES = 2
NUM_SUBCORES = 16
NUM_WORKERS = NUM_CORES * NUM_SUBCORES
REAL_CHUNK = 125
CHUNK = 128
SUPER = 20
LANES = 16
ZCHUNK = 128


def _spmm_sc(x, src4d, dst4d, w4d):
    n, d = x.shape
    n_super = src4d.shape[1]
    n_pad = -(-n // (NUM_SUBCORES * ZCHUNK)) * (NUM_SUBCORES * ZCHUNK)
    rows_per_tile = n_pad // NUM_SUBCORES
    zero_reps = rows_per_tile // CHUNK
    last_rows = n - (NUM_SUBCORES - 1) * rows_per_tile
    segs = d // LANES

    mesh = plsc.VectorSubcoreMesh(core_axis_name="c", subcore_axis_name="s")

    @functools.partial(
        pl.kernel,
        out_type=jax.ShapeDtypeStruct((NUM_CORES, n, d), jnp.float32),
        mesh=mesh,
        scratch_types=[
            pltpu.VMEM((SUPER, CHUNK), jnp.int32),
            pltpu.VMEM((SUPER, CHUNK), jnp.int32),
            pltpu.VMEM((SUPER, CHUNK), jnp.float32),
            pltpu.VMEM((CHUNK, d), jnp.float32),
            pltpu.VMEM_SHARED((n_pad, d), jnp.float32),
            pltpu.SemaphoreType.DMA,
        ],
    )
    def spmm_kernel(x_hbm, src_hbm, dst_hbm, w_hbm, out_hbm,
                    src_s, dst_s, w_s, rows_v, y_acc, sem_i):
        c = lax.axis_index("c")
        s = lax.axis_index("s")
        wid = c * NUM_SUBCORES + s

        zvec = jnp.zeros((LANES,), jnp.float32)

        def zero_body(e, carry):
            for g in range(segs):
                rows_v[e, pl.ds(g * LANES, LANES)] = zvec
            return carry

        lax.fori_loop(0, CHUNK, zero_body, 0)
        for r in range(zero_reps):
            pltpu.sync_copy(
                rows_v, y_acc.at[pl.ds(s * rows_per_tile + r * CHUNK, CHUNK)])
        plsc.subcore_barrier()

        def super_body(sj, carry):
            pltpu.async_copy(src_hbm.at[wid, sj], src_s, sem_i)
            pltpu.async_copy(dst_hbm.at[wid, sj], dst_s, sem_i)
            pltpu.async_copy(w_hbm.at[wid, sj], w_s, sem_i)
            pltpu.make_async_copy(src_hbm.at[wid, sj], src_s, sem_i).wait()
            pltpu.make_async_copy(dst_hbm.at[wid, sj], dst_s, sem_i).wait()
            pltpu.make_async_copy(w_hbm.at[wid, sj], w_s, sem_i).wait()

            def chunk_body(jj, carry1):
                pltpu.sync_copy(x_hbm.at[src_s.at[jj]], rows_v)

                for gg in range(CHUNK // LANES):
                    base = gg * LANES
                    w16 = w_s[jj, pl.ds(base, LANES)]
                    for i in range(LANES):
                        w = w16[i]
                        for g in range(segs):
                            sl = pl.ds(g * LANES, LANES)
                            rows_v[base + i, sl] = rows_v[base + i, sl] * w
                pltpu.sync_copy(rows_v, y_acc.at[dst_s.at[jj]], add=True)
                return carry1

            lax.fori_loop(0, SUPER, chunk_body, 0)
            return carry

        lax.fori_loop(0, n_super, super_body, 0)
        plsc.subcore_barrier()

        base_row = s * rows_per_tile

        @pl.when(s < NUM_SUBCORES - 1)
        def _():
            pltpu.sync_copy(y_acc.at[pl.ds(base_row, rows_per_tile)],
                            out_hbm.at[c, pl.ds(base_row, rows_per_tile)])

        @pl.when(s == NUM_SUBCORES - 1)
        def _():
            last_base = (NUM_SUBCORES - 1) * rows_per_tile
            pltpu.sync_copy(y_acc.at[pl.ds(last_base, last_rows)],
                            out_hbm.at[c, pl.ds(last_base, last_rows)])

    return spmm_kernel(x, src4d, dst4d, w4d)


def _combine_tc(partials):
    _, n, d = partials.shape
    blk = n // 10

    def body(p_ref, o_ref):
        o_ref[...] = p_ref[0] + p_ref[1]

    return pl.pallas_call(
        body,
        out_shape=jax.ShapeDtypeStruct((n, d), jnp.float32),
        grid=(n // blk,),
        in_specs=[pl.BlockSpec((2, blk, d), lambda i: (0, i, 0))],
        out_specs=pl.BlockSpec((blk, d), lambda i: (i, 0)),
    )(partials)


def _project_tc(x0, x1, x2, x3, weights, bias2d):
    n, d = x0.shape
    kp1, _, d_out = weights.shape
    blk = n // 10

    def body(x0_ref, x1_ref, x2_ref, x3_ref, w_ref, b_ref, o_ref):
        acc = jnp.dot(x0_ref[...], w_ref[0], preferred_element_type=jnp.float32)
        acc = acc + jnp.dot(x1_ref[...], w_ref[1], preferred_element_type=jnp.float32)
        acc = acc + jnp.dot(x2_ref[...], w_ref[2], preferred_element_type=jnp.float32)
        acc = acc + jnp.dot(x3_ref[...], w_ref[3], preferred_element_type=jnp.float32)
        o_ref[...] = acc + b_ref[...]

    x_spec = pl.BlockSpec((blk, d), lambda i: (i, 0))
    return pl.pallas_call(
        body,
        out_shape=jax.ShapeDtypeStruct((n, d_out), jnp.float32),
        grid=(n // blk,),
        in_specs=[x_spec, x_spec, x_spec, x_spec,
                  pl.BlockSpec((kp1, d, d_out), lambda i: (0, 0, 0)),
                  pl.BlockSpec((1, d_out), lambda i: (0, 0))],
        out_specs=pl.BlockSpec((blk, d_out), lambda i: (i, 0)),
    )(x0, x1, x2, x3, weights, bias2d)


def _pad_edges(arr, pad_value):
    e = arr.shape[0]
    n_chunks = e // REAL_CHUNK
    a = arr.reshape(n_chunks, REAL_CHUNK)
    a = jnp.pad(a, ((0, 0), (0, CHUNK - REAL_CHUNK)), constant_values=pad_value)
    return a.reshape(NUM_WORKERS, -1, SUPER, CHUNK)


def kernel(x, edge_index, edge_weight, weights, bias):
    n = x.shape[0]
    dst = _pad_edges(edge_index[0].astype(jnp.int32), n)
    src = _pad_edges(edge_index[1].astype(jnp.int32), 0)
    w4d = _pad_edges(edge_weight, 0.0)

    x1 = _combine_tc(_spmm_sc(x, src, dst, w4d))
    x2 = _combine_tc(_spmm_sc(x1, src, dst, w4d))
    x3 = _combine_tc(_spmm_sc(x2, src, dst, w4d))
    return _project_tc(x, x1, x2, x3, weights, bias.reshape(1, -1))


# speedup vs baseline: 1.4057x; 1.4057x over previous
import functools

import jax
import jax.numpy as jnp
from jax import lax
from jax.experimental import pallas as pl
from jax.experimental.pallas import tpu as pltpu
from jax.experimental.pallas import tpu_sc as plsc

NUM_CORES = 2
NUM_SUBCORES = 16
NUM_WORKERS = NUM_CORES * NUM_SUBCORES
CHUNK = 80
SUPER = 25
LANES = 16
ZCHUNK = 128


def _spmm_sc(x, src4d, dst4d, w4d):
    n, d = x.shape
    n_super = src4d.shape[1]
    n_pad = -(-n // (NUM_SUBCORES * ZCHUNK)) * (NUM_SUBCORES * ZCHUNK)
    rows_per_tile = n_pad // NUM_SUBCORES
    zero_reps = rows_per_tile // ZCHUNK
    last_rows = n - (NUM_SUBCORES - 1) * rows_per_tile
    segs = d // LANES

    mesh = plsc.VectorSubcoreMesh(core_axis_name="c", subcore_axis_name="s")

    @functools.partial(
        pl.kernel,
        out_type=jax.ShapeDtypeStruct((NUM_CORES, n, d), jnp.float32),
        mesh=mesh,
        scratch_types=[
            pltpu.VMEM((SUPER, CHUNK), jnp.int32),
            pltpu.VMEM((SUPER, CHUNK), jnp.int32),
            pltpu.VMEM((SUPER, CHUNK), jnp.float32),
            pltpu.VMEM((CHUNK, d), jnp.float32),
            pltpu.VMEM_SHARED((n_pad, d), jnp.float32),
            pltpu.SemaphoreType.DMA,
        ],
    )
    def spmm_kernel(x_hbm, src_hbm, dst_hbm, w_hbm, out_hbm,
                    src_s, dst_s, w_s, rows_v, y_acc, sem_i):
        c = lax.axis_index("c")
        s = lax.axis_index("s")
        wid = c * NUM_SUBCORES + s
        base_row = s * rows_per_tile
        last_base = (NUM_SUBCORES - 1) * rows_per_tile

        zvec = jnp.zeros((LANES,), jnp.float32)

        def zero_body(e, carry):
            for g in range(segs):
                rows_v[e, pl.ds(g * LANES, LANES)] = zvec
            return carry

        lax.fori_loop(0, CHUNK, zero_body, 0)
        for r in range(rows_per_tile // CHUNK):
            pltpu.sync_copy(
                rows_v, y_acc.at[pl.ds(base_row + r * CHUNK, CHUNK)])
        plsc.subcore_barrier()

        def super_body(sj, carry):
            pltpu.async_copy(src_hbm.at[wid, sj], src_s, sem_i)
            pltpu.async_copy(dst_hbm.at[wid, sj], dst_s, sem_i)
            pltpu.async_copy(w_hbm.at[wid, sj], w_s, sem_i)
            pltpu.make_async_copy(src_hbm.at[wid, sj], src_s, sem_i).wait()
            pltpu.make_async_copy(dst_hbm.at[wid, sj], dst_s, sem_i).wait()
            pltpu.make_async_copy(w_hbm.at[wid, sj], w_s, sem_i).wait()

            def chunk_body(jj, carry1):
                pltpu.sync_copy(x_hbm.at[src_s.at[jj]], rows_v)
                for gg in range(CHUNK // LANES):
                    base = gg * LANES
                    w16 = w_s[jj, pl.ds(base, LANES)]
                    for i in range(LANES):
                        w = w16[i]
                        for g in range(segs):
                            sl = pl.ds(g * LANES, LANES)
                            rows_v[base + i, sl] = rows_v[base + i, sl] * w
                pltpu.sync_copy(rows_v, y_acc.at[dst_s.at[jj]], add=True)
                return carry1

            lax.fori_loop(0, SUPER, chunk_body, 0)
            return carry

        lax.fori_loop(0, n_super, super_body, 0)
        plsc.subcore_barrier()

        @pl.when(s < NUM_SUBCORES - 1)
        def _():
            pltpu.sync_copy(y_acc.at[pl.ds(base_row, rows_per_tile)],
                            out_hbm.at[c, pl.ds(base_row, rows_per_tile)])

        @pl.when(s == NUM_SUBCORES - 1)
        def _():
            pltpu.sync_copy(y_acc.at[pl.ds(last_base, last_rows)],
                            out_hbm.at[c, pl.ds(last_base, last_rows)])

    return spmm_kernel(x, src4d, dst4d, w4d)


def _combine_tc(partials):
    _, n, d = partials.shape
    blk = n // 10

    def body(p_ref, o_ref):
        o_ref[...] = p_ref[0] + p_ref[1]

    return pl.pallas_call(
        body,
        out_shape=jax.ShapeDtypeStruct((n, d), jnp.float32),
        grid=(n // blk,),
        in_specs=[pl.BlockSpec((2, blk, d), lambda i: (0, i, 0))],
        out_specs=pl.BlockSpec((blk, d), lambda i: (i, 0)),
    )(partials)


def _project_tc(x0, x1, x2, x3, weights, bias2d):
    n, d = x0.shape
    kp1, _, d_out = weights.shape
    blk = n // 10

    def body(x0_ref, x1_ref, x2_ref, x3_ref, w_ref, b_ref, o_ref):
        acc = jnp.dot(x0_ref[...], w_ref[0], preferred_element_type=jnp.float32)
        acc = acc + jnp.dot(x1_ref[...], w_ref[1], preferred_element_type=jnp.float32)
        acc = acc + jnp.dot(x2_ref[...], w_ref[2], preferred_element_type=jnp.float32)
        acc = acc + jnp.dot(x3_ref[...], w_ref[3], preferred_element_type=jnp.float32)
        o_ref[...] = acc + b_ref[...]

    x_spec = pl.BlockSpec((blk, d), lambda i: (i, 0))
    return pl.pallas_call(
        body,
        out_shape=jax.ShapeDtypeStruct((n, d_out), jnp.float32),
        grid=(n // blk,),
        in_specs=[x_spec, x_spec, x_spec, x_spec,
                  pl.BlockSpec((kp1, d, d_out), lambda i: (0, 0, 0)),
                  pl.BlockSpec((1, d_out), lambda i: (0, 0))],
        out_specs=pl.BlockSpec((blk, d_out), lambda i: (i, 0)),
    )(x0, x1, x2, x3, weights, bias2d)


def kernel(x, edge_index, edge_weight, weights, bias):
    dst = edge_index[0].astype(jnp.int32).reshape(NUM_WORKERS, -1, SUPER, CHUNK)
    src = edge_index[1].astype(jnp.int32).reshape(NUM_WORKERS, -1, SUPER, CHUNK)
    w4d = edge_weight.reshape(NUM_WORKERS, -1, SUPER, CHUNK)

    x1 = _combine_tc(_spmm_sc(x, src, dst, w4d))
    x2 = _combine_tc(_spmm_sc(x1, src, dst, w4d))
    x3 = _combine_tc(_spmm_sc(x2, src, dst, w4d))
    return _project_tc(x, x1, x2, x3, weights, bias.reshape(1, -1))
